# Initial kernel scaffold; baseline (speedup 1.0000x reference)
import jax
import jax.numpy as jnp
from jax.experimental import pallas as pl


def kernel(x, table):
    raise NotImplementedError("write your pallas kernel here")


# speedup vs baseline: 2.8990x; 2.8990x over previous
import functools
import math

import jax
import jax.numpy as jnp
from jax import lax
from jax.experimental import pallas as pl
from jax.experimental.pallas import tpu as pltpu
from jax.experimental.pallas import tpu_sc as plsc

D_EMBED = 64
SCALE = math.sqrt(D_EMBED)

NC, NS = 2, 16
NW = NC * NS
CH = 128


def _make_kernel(B):
    assert B % (NW * CH) == 0
    n_chunks = B // (NW * CH)
    mesh = plsc.VectorSubcoreMesh(
        core_axis_name="c", subcore_axis_name="s",
        num_cores=NC, num_subcores=NS)

    @functools.partial(
        pl.kernel,
        out_type=jax.ShapeDtypeStruct((B, D_EMBED), jnp.float32),
        mesh=mesh,
        scratch_types=[
            pltpu.VMEM((n_chunks, CH), jnp.int32),
            pltpu.VMEM((CH, D_EMBED), jnp.float32),
            pltpu.SemaphoreType.DMA,
        ],
        compiler_params=pltpu.CompilerParams(use_tc_tiling_on_sc=False),
    )
    def k(x_hbm, table_hbm, out_hbm, idx_v, rows_v, sem):
        wid = lax.axis_index("s") * NC + lax.axis_index("c")
        row0 = wid * n_chunks
        pltpu.sync_copy(x_hbm.at[wid], idx_v)

        def chunk(j, carry):
            pltpu.async_copy(table_hbm.at[idx_v.at[j]], rows_v, sem).wait()

            def scale_row(i, c):
                for p in range(D_EMBED // 16):
                    rows_v[i, pl.ds(p * 16, 16)] = (
                        rows_v[i, pl.ds(p * 16, 16)] * SCALE)
                return c
            lax.fori_loop(0, CH, scale_row, 0)

            pltpu.sync_copy(rows_v, out_hbm.at[pl.ds((row0 + j) * CH, CH)])
            return carry
        lax.fori_loop(0, n_chunks, chunk, 0)

    return k


def kernel(x, table):
    B = x.shape[0] * x.shape[1]
    x2d = x.reshape(NW, B // (NW * CH), CH).astype(jnp.int32)
    out = _make_kernel(B)(x2d, table)
    return out.reshape(x.shape[0], x.shape[1], D_EMBED)


# speedup vs baseline: 3.4749x; 1.1987x over previous
import functools
import math

import jax
import jax.numpy as jnp
from jax import lax
from jax.experimental import pallas as pl
from jax.experimental.pallas import tpu as pltpu
from jax.experimental.pallas import tpu_sc as plsc

D_EMBED = 64
SCALE = math.sqrt(D_EMBED)

NC, NS = 2, 16
NW = NC * NS
CH = 128
ROWS_PER_IT = 8


def _make_kernel(B):
    assert B % (NW * CH) == 0
    n_chunks = B // (NW * CH)
    assert n_chunks % 2 == 0
    mesh = plsc.VectorSubcoreMesh(
        core_axis_name="c", subcore_axis_name="s",
        num_cores=NC, num_subcores=NS)

    @functools.partial(
        pl.kernel,
        out_type=jax.ShapeDtypeStruct((B, D_EMBED), jnp.float32),
        mesh=mesh,
        scratch_types=[
            pltpu.VMEM((n_chunks, CH), jnp.int32),
            pltpu.VMEM((CH, D_EMBED), jnp.float32),
            pltpu.VMEM((CH, D_EMBED), jnp.float32),
            pltpu.SemaphoreType.DMA((2,)),
            pltpu.SemaphoreType.DMA((2,)),
        ],
        compiler_params=pltpu.CompilerParams(use_tc_tiling_on_sc=False),
    )
    def k(x_hbm, table_hbm, out_hbm, idx_v, rows0, rows1, gsem, ssem):
        wid = lax.axis_index("s") * NC + lax.axis_index("c")
        row0 = wid * n_chunks
        pltpu.sync_copy(x_hbm.at[wid], idx_v)
        bufs = (rows0, rows1)

        def gather_start(g, b):
            pltpu.async_copy(table_hbm.at[idx_v.at[g]], bufs[b], gsem.at[b])

        def gather_wait(b):
            pltpu.make_async_copy(
                table_hbm.at[idx_v.at[0]], bufs[b], gsem.at[b]).wait()

        def store_start(g, b):
            pltpu.async_copy(
                bufs[b], out_hbm.at[pl.ds((row0 + g) * CH, CH)], ssem.at[b])

        def store_wait(b):
            pltpu.make_async_copy(
                bufs[b], out_hbm.at[pl.ds(row0 * CH, CH)], ssem.at[b]).wait()

        def scale(b):
            buf = bufs[b]

            def body(i, c):
                for r in range(ROWS_PER_IT):
                    for p in range(D_EMBED // 16):
                        sl = (i * ROWS_PER_IT + r, pl.ds(p * 16, 16))
                        buf[sl] = buf[sl] * SCALE
                return c
            lax.fori_loop(0, CH // ROWS_PER_IT, body, 0)

        gather_start(0, 0)

        def pair(t, c):
            for ph in range(2):
                g = 2 * t + ph
                b, nb = ph, 1 - ph

                @pl.when(jnp.logical_and(g >= 1, g + 1 < n_chunks))
                def _():
                    store_wait(nb)

                @pl.when(g + 1 < n_chunks)
                def _():
                    gather_start(g + 1, nb)

                gather_wait(b)
                scale(b)
                store_start(g, b)
            return c
        lax.fori_loop(0, n_chunks // 2, pair, 0)
        store_wait(0)
        store_wait(1)

    return k


def kernel(x, table):
    B = x.shape[0] * x.shape[1]
    x3d = x.reshape(NW, B // (NW * CH), CH).astype(jnp.int32)
    out = _make_kernel(B)(x3d, table)
    return out.reshape(x.shape[0], x.shape[1], D_EMBED)
